# baseline (speedup 1.0000x reference)
import jax
import jax.numpy as jnp
from jax.experimental import pallas as pl
from jax.experimental.pallas import tpu as pltpu

_IDX0 = 7
_STRIDE = 156
_NPAIRS = 64
_NUM_NODES = 10000
_W = 2 * _NUM_NODES

_BLOCK_ROWS = 64


def _insert_pairs(scr, x):
    for k in range(_NPAIRS):
        col = 2 * (_IDX0 + _STRIDE * k)
        scr[:, col:col + 2] = x[:, 2 * k:2 * k + 2]


def _body(x_ref, o_ref, scr0, scr1, sem0, sem1):
    i = pl.program_id(0)
    n = pl.num_programs(0)
    r = _BLOCK_ROWS

    @pl.when(i == 0)
    def _():
        scr0[...] = jnp.zeros_like(scr0)
        scr1[...] = jnp.zeros_like(scr1)

    def step(scr, sem):
        @pl.when(i >= 2)
        def _():
            pltpu.make_async_copy(
                scr, o_ref.at[pl.ds((i - 2) * r, r), :], sem).wait()

        _insert_pairs(scr, x_ref[...])
        pltpu.make_async_copy(
            scr, o_ref.at[pl.ds(i * r, r), :], sem).start()

    @pl.when(i % 2 == 0)
    def _():
        step(scr0, sem0)

    @pl.when(i % 2 == 1)
    def _():
        step(scr1, sem1)

    @pl.when(i == n - 1)
    def _():
        pltpu.make_async_copy(
            scr0, o_ref.at[pl.ds((n - 2) * r, r), :], sem0).wait()
        pltpu.make_async_copy(
            scr1, o_ref.at[pl.ds((n - 1) * r, r), :], sem1).wait()


def kernel(f_lat):
    rows = f_lat.shape[0]
    out = pl.pallas_call(
        _body,
        grid=(rows // _BLOCK_ROWS,),
        in_specs=[pl.BlockSpec((_BLOCK_ROWS, 128), lambda i: (i, 0))],
        out_specs=pl.BlockSpec(memory_space=pl.ANY),
        out_shape=jax.ShapeDtypeStruct((rows, _W), f_lat.dtype),
        scratch_shapes=[
            pltpu.VMEM((_BLOCK_ROWS, _W), jnp.float32),
            pltpu.VMEM((_BLOCK_ROWS, _W), jnp.float32),
            pltpu.SemaphoreType.DMA,
            pltpu.SemaphoreType.DMA,
        ],
    )(f_lat)
    return out.reshape(rows, _NUM_NODES, 2)
